# baseline (speedup 1.0000x reference)
import jax
import jax.numpy as jnp
from jax.experimental import pallas as pl
from jax.experimental.pallas import tpu as pltpu

NCODES = 1000
CPAD = 1024
DIM = 20
GROWS = 32
TB = 512


def _vq_body(x_ref, em_ref, e2_ref, g_ref, q_ref, idx_ref, hist_ref, loss_ref):
    step = pl.program_id(0)
    xb = x_ref[0]

    dot = jax.lax.dot_general(
        em_ref[...], xb, (((1,), (0,)), ((), ())),
        preferred_element_type=jnp.float32)
    dist = dot + e2_ref[...]

    minv = jnp.min(dist, axis=0, keepdims=True)
    riota = jax.lax.broadcasted_iota(jnp.int32, (CPAD, TB), 0)
    idx = jnp.min(jnp.where(dist <= minv, riota, CPAD), axis=0)
    onehot = (riota == idx[None, :]).astype(jnp.float32)

    ext = jax.lax.dot_general(
        g_ref[...], onehot, (((1,), (0,)), ((), ())),
        preferred_element_type=jnp.float32)
    qT = ext[:DIM]
    q_ref[0] = qT
    idx_ref[0, 0] = idx

    mx_i = jnp.max(xb, axis=0, keepdims=True)
    ex = jnp.exp(xb - mx_i)
    se = jnp.sum(ex, axis=0, keepdims=True)
    sm_i = ex / se
    log_sm_i = (xb - mx_i) - jnp.log(se)
    mx_q = jnp.max(qT, axis=0, keepdims=True)
    eq = jnp.exp(qT - mx_q)
    sm_q = eq / jnp.sum(eq, axis=0, keepdims=True)
    tile_loss = jnp.sum(sm_i * (log_sm_i - sm_q)).reshape(1, 1)

    @pl.when(step == 0)
    def _init():
        hist_ref[...] = jnp.zeros_like(hist_ref)
        loss_ref[...] = jnp.zeros_like(loss_ref)

    hist_ref[...] += jnp.sum(onehot, axis=1, keepdims=True)
    loss_ref[...] += tile_loss


def kernel(inputs, emb_w):
    B, C, T = inputs.shape
    nt = T // TB
    ng = B * nt
    ewp = jnp.zeros((CPAD, DIM), jnp.float32).at[:NCODES].set(emb_w)
    e2 = jnp.full((CPAD, 1), 1e30, jnp.float32).at[:NCODES, 0].set(
        jnp.sum(emb_w * emb_w, axis=1))
    em = -2.0 * ewp
    g = jnp.zeros((GROWS, CPAD), jnp.float32).at[:DIM].set(ewp.T)

    q, idxo, hist, loss_sum = pl.pallas_call(
        _vq_body,
        grid=(ng,),
        in_specs=[
            pl.BlockSpec((1, DIM, TB), lambda i: (i // nt, 0, i % nt)),
            pl.BlockSpec((CPAD, DIM), lambda i: (0, 0)),
            pl.BlockSpec((CPAD, 1), lambda i: (0, 0)),
            pl.BlockSpec((GROWS, CPAD), lambda i: (0, 0)),
        ],
        out_specs=[
            pl.BlockSpec((1, DIM, TB), lambda i: (i // nt, 0, i % nt)),
            pl.BlockSpec((1, 1, TB), lambda i: (i, 0, 0)),
            pl.BlockSpec((CPAD, 1), lambda i: (0, 0)),
            pl.BlockSpec((1, 1), lambda i: (0, 0)),
        ],
        out_shape=[
            jax.ShapeDtypeStruct((B, C, T), jnp.float32),
            jax.ShapeDtypeStruct((ng, 1, TB), jnp.int32),
            jax.ShapeDtypeStruct((CPAD, 1), jnp.float32),
            jax.ShapeDtypeStruct((1, 1), jnp.float32),
        ],
        compiler_params=pltpu.CompilerParams(
            dimension_semantics=("arbitrary",)),
    )(inputs, em, e2, g)

    enc_idx = idxo.reshape(-1)
    avg = hist[:NCODES, 0] / (B * T)
    perplexity = jnp.exp(-jnp.sum(avg * jnp.log(avg + 1e-10)))
    loss = 0.1 * loss_sum[0, 0] / B
    return q, loss, perplexity, emb_w, enc_idx


# speedup vs baseline: 1.9272x; 1.9272x over previous
import jax
import jax.numpy as jnp
from jax.experimental import pallas as pl
from jax.experimental.pallas import tpu as pltpu

NCODES = 1000
CPAD = 1024
DIM = 20
GROWS = 32
TB = 4096


def _vq_body(x_ref, em_ref, e2_ref, g_ref, q_ref, idx_ref, hist_ref, loss_ref):
    step = pl.program_id(0)
    xb = x_ref[0]

    dot = jax.lax.dot_general(
        em_ref[...], xb, (((1,), (0,)), ((), ())),
        preferred_element_type=jnp.float32)
    dist = dot + e2_ref[...]

    riota = jax.lax.broadcasted_iota(jnp.int32, (CPAD, TB), 0)
    idx = jnp.argmin(dist, axis=0).astype(jnp.int32)
    onehot = (riota == idx[None, :]).astype(jnp.float32)

    ext = jax.lax.dot_general(
        g_ref[...], onehot, (((1,), (0,)), ((), ())),
        preferred_element_type=jnp.float32)
    qT = ext[:DIM]
    q_ref[0] = qT
    idx_ref[0, 0] = idx

    mx_i = jnp.max(xb, axis=0, keepdims=True)
    ex = jnp.exp(xb - mx_i)
    se = jnp.sum(ex, axis=0, keepdims=True)
    sm_i = ex / se
    log_sm_i = (xb - mx_i) - jnp.log(se)
    mx_q = jnp.max(qT, axis=0, keepdims=True)
    eq = jnp.exp(qT - mx_q)
    sm_q = eq / jnp.sum(eq, axis=0, keepdims=True)
    tile_loss = jnp.sum(sm_i * (log_sm_i - sm_q)).reshape(1, 1)

    @pl.when(step == 0)
    def _init():
        hist_ref[...] = jnp.zeros_like(hist_ref)
        loss_ref[...] = jnp.zeros_like(loss_ref)

    hist_ref[...] += jnp.sum(onehot, axis=1, keepdims=True)
    loss_ref[...] += tile_loss


def kernel(inputs, emb_w):
    B, C, T = inputs.shape
    nt = T // TB
    ng = B * nt
    ewp = jnp.zeros((CPAD, DIM), jnp.float32).at[:NCODES].set(emb_w)
    e2 = jnp.full((CPAD, 1), 1e30, jnp.float32).at[:NCODES, 0].set(
        jnp.sum(emb_w * emb_w, axis=1))
    em = -2.0 * ewp
    g = jnp.zeros((GROWS, CPAD), jnp.float32).at[:DIM].set(ewp.T)

    q, idxo, hist, loss_sum = pl.pallas_call(
        _vq_body,
        grid=(ng,),
        in_specs=[
            pl.BlockSpec((1, DIM, TB), lambda i: (i // nt, 0, i % nt)),
            pl.BlockSpec((CPAD, DIM), lambda i: (0, 0)),
            pl.BlockSpec((CPAD, 1), lambda i: (0, 0)),
            pl.BlockSpec((GROWS, CPAD), lambda i: (0, 0)),
        ],
        out_specs=[
            pl.BlockSpec((1, DIM, TB), lambda i: (i // nt, 0, i % nt)),
            pl.BlockSpec((1, 1, TB), lambda i: (i, 0, 0)),
            pl.BlockSpec((CPAD, 1), lambda i: (0, 0)),
            pl.BlockSpec((1, 1), lambda i: (0, 0)),
        ],
        out_shape=[
            jax.ShapeDtypeStruct((B, C, T), jnp.float32),
            jax.ShapeDtypeStruct((ng, 1, TB), jnp.int32),
            jax.ShapeDtypeStruct((CPAD, 1), jnp.float32),
            jax.ShapeDtypeStruct((1, 1), jnp.float32),
        ],
        compiler_params=pltpu.CompilerParams(
            dimension_semantics=("arbitrary",)),
    )(inputs, em, e2, g)

    enc_idx = idxo.reshape(-1)
    avg = hist[:NCODES, 0] / (B * T)
    perplexity = jnp.exp(-jnp.sum(avg * jnp.log(avg + 1e-10)))
    loss = 0.1 * loss_sum[0, 0] / B
    return q, loss, perplexity, emb_w, enc_idx
